# Initial kernel scaffold; baseline (speedup 1.0000x reference)
import jax
import jax.numpy as jnp
from jax.experimental import pallas as pl


def kernel(x, edge_index, W1, b1, W2, b2, W3, b3, Wc1, bc1, Wc2, bc2):
    raise NotImplementedError("write your pallas kernel here")


# speedup vs baseline: 9.4434x; 9.4434x over previous
import functools

import jax
import jax.numpy as jnp
from jax import lax
from jax.experimental import pallas as pl
from jax.experimental.pallas import tpu as pltpu
from jax.experimental.pallas import tpu_sc as plsc

N = 10000
P = 8
CIN = 512
HID = 512
COUT = 256
HALF = COUT // 2
E = 160000
C = 125
NCH = E // C
NSC = 2
NT = 16
N_PAD = 10240
STRIPE = N_PAD // NT


def _vs_mesh():
    return plsc.VectorSubcoreMesh(core_axis_name="c", subcore_axis_name="s")


def _sc_scatter(z2d, srcoff2d, dst2d):
    cpt = NCH // NT

    @functools.partial(
        pl.kernel,
        out_type=jax.ShapeDtypeStruct((NSC * N_PAD, HALF), jnp.float32),
        mesh=_vs_mesh(),
        scratch_types=[
            pltpu.VMEM((cpt, C), jnp.int32),
            pltpu.VMEM((cpt, C), jnp.int32),
            pltpu.VMEM((C, HALF), jnp.float32),
            pltpu.VMEM_SHARED((N_PAD, HALF), jnp.float32),
            pltpu.SemaphoreType.DMA,
        ],
    )
    def scat_kernel(z_hbm, src_hbm, dst_hbm, out_hbm,
                    src_v, dst_v, buf, acc, sem):
        c = lax.axis_index("c")
        s = lax.axis_index("s")
        base_n = s * STRIPE
        pltpu.sync_copy(z_hbm.at[pl.ds(c * N_PAD + base_n, STRIPE)],
                        acc.at[pl.ds(base_n, STRIPE)])
        pltpu.sync_copy(src_hbm.at[pl.ds(c * NCH + s * cpt, cpt)], src_v)
        pltpu.sync_copy(dst_hbm.at[pl.ds(s * cpt, cpt)], dst_v)
        plsc.subcore_barrier()

        def body(j, carry):
            pltpu.async_copy(z_hbm.at[src_v.at[j]], buf, sem).wait()
            pltpu.sync_copy(buf, acc.at[dst_v.at[j]], add=True)
            return carry

        lax.fori_loop(0, cpt, body, 0)
        plsc.subcore_barrier()
        pltpu.sync_copy(acc.at[pl.ds(base_n, STRIPE)],
                        out_hbm.at[pl.ds(c * N_PAD + base_n, STRIPE)])

    return scat_kernel(z2d, srcoff2d, dst2d)


EMB_NB = 200
GC_NB = 1000


def _dinv_col(degp_ref):
    return lax.rsqrt(degp_ref[:, 0:1])


def _embed_body(x_ref, w1_ref, b1_ref, w2_ref, b2_ref, w3_ref, b3_ref, o_ref):
    xb = x_ref[...].reshape(EMB_NB * P, CIN)
    t = jnp.dot(xb, w1_ref[...], preferred_element_type=jnp.float32)
    t = jnp.maximum(t + b1_ref[...], 0.0)
    sp = jnp.sum(t.reshape(EMB_NB, P, HID), axis=1)
    u = jnp.dot(sp, w2_ref[...], preferred_element_type=jnp.float32)
    u = u + float(P) * b2_ref[...]
    v = jnp.dot(u, w3_ref[...], preferred_element_type=jnp.float32)
    o_ref[...] = jnp.maximum(v + b3_ref[...], 0.0)


def _embed(x, W1, b1, W2, b2, W3, b3):
    grid = N // EMB_NB
    return pl.pallas_call(
        _embed_body,
        grid=(grid,),
        in_specs=[
            pl.BlockSpec((EMB_NB, P, CIN), lambda i: (i, 0, 0)),
            pl.BlockSpec((CIN, HID), lambda i: (0, 0)),
            pl.BlockSpec((1, HID), lambda i: (0, 0)),
            pl.BlockSpec((HID, COUT), lambda i: (0, 0)),
            pl.BlockSpec((1, COUT), lambda i: (0, 0)),
            pl.BlockSpec((COUT, COUT), lambda i: (0, 0)),
            pl.BlockSpec((1, COUT), lambda i: (0, 0)),
        ],
        out_specs=pl.BlockSpec((EMB_NB, COUT), lambda i: (i, 0)),
        out_shape=jax.ShapeDtypeStruct((N, COUT), jnp.float32),
    )(x, W1, b1[None], W2, b2[None], W3, b3[None])


def _pre_body(h_ref, w_ref, degp_ref, o_ref):
    dinv = _dinv_col(degp_ref)
    xw = jnp.dot(h_ref[...], w_ref[...], preferred_element_type=jnp.float32)
    z = xw * dinv
    o_ref[0] = z[:, :HALF]
    o_ref[1] = z[:, HALF:]


def _pre(h, Wc1, degp):
    grid = N // GC_NB
    return pl.pallas_call(
        _pre_body,
        grid=(grid,),
        in_specs=[
            pl.BlockSpec((GC_NB, COUT), lambda i: (i, 0)),
            pl.BlockSpec((COUT, COUT), lambda i: (0, 0)),
            pl.BlockSpec((GC_NB, HALF), lambda i: (i, 0)),
        ],
        out_specs=pl.BlockSpec((NSC, GC_NB, HALF), lambda i: (0, i, 0)),
        out_shape=jax.ShapeDtypeStruct((NSC, N, HALF), jnp.float32),
    )(h, Wc1, degp)


def _mid_body(acc_ref, degp_ref, w_ref, b_ref, o_ref):
    dinv = _dinv_col(degp_ref)
    a = jnp.concatenate([acc_ref[0], acc_ref[1]], axis=1)
    y = jnp.maximum(a * dinv + b_ref[...], 0.0)
    xw = jnp.dot(y, w_ref[...], preferred_element_type=jnp.float32)
    z = xw * dinv
    o_ref[0] = z[:, :HALF]
    o_ref[1] = z[:, HALF:]


def _mid(acc1, degp, Wc2, bc1):
    grid = N // GC_NB
    return pl.pallas_call(
        _mid_body,
        grid=(grid,),
        in_specs=[
            pl.BlockSpec((NSC, GC_NB, HALF), lambda i: (0, i, 0)),
            pl.BlockSpec((GC_NB, HALF), lambda i: (i, 0)),
            pl.BlockSpec((COUT, COUT), lambda i: (0, 0)),
            pl.BlockSpec((1, COUT), lambda i: (0, 0)),
        ],
        out_specs=pl.BlockSpec((NSC, GC_NB, HALF), lambda i: (0, i, 0)),
        out_shape=jax.ShapeDtypeStruct((NSC, N, HALF), jnp.float32),
    )(acc1, degp, Wc2, bc1[None])


def _fin_body(acc_ref, degp_ref, b_ref, h_ref, o_ref):
    dinv = _dinv_col(degp_ref)
    y = jnp.concatenate([acc_ref[0], acc_ref[1]], axis=1) * dinv + b_ref[...]
    hh = h_ref[...]
    o_ref[...] = jnp.maximum(hh + y, 0.0) + hh


def _fin(acc2, degp, bc2, h):
    grid = N // GC_NB
    return pl.pallas_call(
        _fin_body,
        grid=(grid,),
        in_specs=[
            pl.BlockSpec((NSC, GC_NB, HALF), lambda i: (0, i, 0)),
            pl.BlockSpec((GC_NB, HALF), lambda i: (i, 0)),
            pl.BlockSpec((1, COUT), lambda i: (0, 0)),
            pl.BlockSpec((GC_NB, COUT), lambda i: (i, 0)),
        ],
        out_specs=pl.BlockSpec((GC_NB, COUT), lambda i: (i, 0)),
        out_shape=jax.ShapeDtypeStruct((N, COUT), jnp.float32),
    )(acc2, degp, bc2[None], h)


def _pad_nodes(z):
    return jnp.pad(z, ((0, 0), (0, N_PAD - N), (0, 0))).reshape(NSC * N_PAD, HALF)


def _unpad_nodes(a):
    return a.reshape(NSC, N_PAD, HALF)[:, :N]


def kernel(x, edge_index, W1, b1, W2, b2, W3, b3, Wc1, bc1, Wc2, bc2):
    src = edge_index[0]
    dst = edge_index[1]
    dst2d = dst.reshape(NCH, C)
    srcoff2d = jnp.concatenate([src, src + N_PAD]).reshape(2 * NCH, C)

    ones2d = _pad_nodes(jnp.ones((NSC, N, HALF), jnp.float32))
    degp = _sc_scatter(ones2d, srcoff2d, dst2d).reshape(
        NSC, N_PAD, HALF)[0, :N]
    h = _embed(x, W1, b1, W2, b2, W3, b3)
    z1 = _pre(h, Wc1, degp)
    acc1 = _unpad_nodes(_sc_scatter(_pad_nodes(z1), srcoff2d, dst2d))
    z2 = _mid(acc1, degp, Wc2, bc1)
    acc2 = _unpad_nodes(_sc_scatter(_pad_nodes(z2), srcoff2d, dst2d))
    return _fin(acc2, degp, bc2, h)


# speedup vs baseline: 11.4564x; 1.2132x over previous
import functools

import jax
import jax.numpy as jnp
from jax import lax
from jax.experimental import pallas as pl
from jax.experimental.pallas import tpu as pltpu
from jax.experimental.pallas import tpu_sc as plsc

N = 10000
P = 8
CIN = 512
HID = 512
COUT = 256
HALF = COUT // 2
E = 160000
C = 125
NCH = E // C
NSC = 2
NT = 16
N_PAD = 10240
STRIPE = N_PAD // NT
NBUF = 2
NPHASE = 2


def _vs_mesh():
    return plsc.VectorSubcoreMesh(core_axis_name="c", subcore_axis_name="s")


def _sc_scatter(z2d, srcoff2d, dst2d):
    cpt = NCH // NT
    cpp = cpt // NPHASE

    @functools.partial(
        pl.kernel,
        out_type=jax.ShapeDtypeStruct((NSC * N_PAD, HALF), jnp.float32),
        mesh=_vs_mesh(),
        scratch_types=[
            pltpu.VMEM((cpp, C), jnp.int32),
            pltpu.VMEM((cpp, C), jnp.int32),
            pltpu.VMEM((C, HALF), jnp.float32),
            pltpu.VMEM((C, HALF), jnp.float32),
            pltpu.VMEM_SHARED((N_PAD, HALF), jnp.float32),
            pltpu.SemaphoreType.DMA((NBUF,)),
            pltpu.SemaphoreType.DMA((NBUF,)),
        ],
    )
    def scat_kernel(z_hbm, src_hbm, dst_hbm, out_hbm,
                    src_v, dst_v, buf0, buf1, acc, gsem, ssem):
        c = lax.axis_index("c")
        s = lax.axis_index("s")
        base_n = s * STRIPE
        bufs = [buf0, buf1]
        pltpu.sync_copy(z_hbm.at[pl.ds(c * N_PAD + base_n, STRIPE)],
                        acc.at[pl.ds(base_n, STRIPE)])
        plsc.subcore_barrier()

        for ph in range(NPHASE):
            pltpu.sync_copy(
                src_hbm.at[pl.ds(c * NCH + s * cpt + ph * cpp, cpp)], src_v)
            pltpu.sync_copy(
                dst_hbm.at[pl.ds(s * cpt + ph * cpp, cpp)], dst_v)

            for b in range(NBUF):
                pltpu.async_copy(z_hbm.at[src_v.at[b]], bufs[b], gsem.at[b])

            def round_body(g, carry):
                for b in range(NBUF):
                    j = g * NBUF + b
                    pltpu.make_async_copy(z_hbm.at[src_v.at[j]], bufs[b],
                                          gsem.at[b]).wait()
                    pltpu.async_copy(bufs[b], acc.at[dst_v.at[j]],
                                     ssem.at[b], add=True)
                for b in range(NBUF):
                    j = g * NBUF + b
                    jn = j + NBUF
                    pltpu.make_async_copy(bufs[b], acc.at[dst_v.at[j]],
                                          ssem.at[b]).wait()

                    @pl.when(jn < cpp)
                    def _():
                        pltpu.async_copy(z_hbm.at[src_v.at[jn]], bufs[b],
                                         gsem.at[b])
                return carry

            lax.fori_loop(0, cpp // NBUF, round_body, 0)
        plsc.subcore_barrier()
        pltpu.sync_copy(acc.at[pl.ds(base_n, STRIPE)],
                        out_hbm.at[pl.ds(c * N_PAD + base_n, STRIPE)])

    return scat_kernel(z2d, srcoff2d, dst2d)


EMB_NB = 200
GC_NB = 1024


def _dinv_col(degp_ref):
    return lax.rsqrt(jnp.maximum(degp_ref[0, :, 0:1], 1.0))


def _embed_body(x_ref, w1_ref, b1_ref, w2_ref, b2_ref, w3_ref, b3_ref, o_ref):
    xb = x_ref[...].reshape(EMB_NB * P, CIN)
    t = jnp.dot(xb, w1_ref[...], preferred_element_type=jnp.float32)
    t = jnp.maximum(t + b1_ref[...], 0.0)
    sp = jnp.sum(t.reshape(EMB_NB, P, HID), axis=1)
    u = jnp.dot(sp, w2_ref[...], preferred_element_type=jnp.float32)
    u = u + float(P) * b2_ref[...]
    v = jnp.dot(u, w3_ref[...], preferred_element_type=jnp.float32)
    o_ref[...] = jnp.maximum(v + b3_ref[...], 0.0)


def _embed(x, W1, b1, W2, b2, W3, b3):
    grid = N // EMB_NB
    return pl.pallas_call(
        _embed_body,
        grid=(grid,),
        in_specs=[
            pl.BlockSpec((EMB_NB, P, CIN), lambda i: (i, 0, 0)),
            pl.BlockSpec((CIN, HID), lambda i: (0, 0)),
            pl.BlockSpec((1, HID), lambda i: (0, 0)),
            pl.BlockSpec((HID, COUT), lambda i: (0, 0)),
            pl.BlockSpec((1, COUT), lambda i: (0, 0)),
            pl.BlockSpec((COUT, COUT), lambda i: (0, 0)),
            pl.BlockSpec((1, COUT), lambda i: (0, 0)),
        ],
        out_specs=pl.BlockSpec((EMB_NB, COUT), lambda i: (i, 0)),
        out_shape=jax.ShapeDtypeStruct((N, COUT), jnp.float32),
    )(x, W1, b1[None], W2, b2[None], W3, b3[None])


def _pre_body(h_ref, w_ref, degp_ref, o_ref):
    dinv = _dinv_col(degp_ref)
    xw = jnp.dot(h_ref[...], w_ref[...], preferred_element_type=jnp.float32)
    z = xw * dinv
    o_ref[0] = z[:, :HALF]
    o_ref[1] = z[:, HALF:]


def _pre(h_pad, Wc1, degp):
    grid = N_PAD // GC_NB
    return pl.pallas_call(
        _pre_body,
        grid=(grid,),
        in_specs=[
            pl.BlockSpec((GC_NB, COUT), lambda i: (i, 0)),
            pl.BlockSpec((COUT, COUT), lambda i: (0, 0)),
            pl.BlockSpec((NSC, GC_NB, HALF), lambda i: (0, i, 0)),
        ],
        out_specs=pl.BlockSpec((NSC, GC_NB, HALF), lambda i: (0, i, 0)),
        out_shape=jax.ShapeDtypeStruct((NSC, N_PAD, HALF), jnp.float32),
    )(h_pad, Wc1, degp)


def _mid_body(acc_ref, degp_ref, w_ref, b_ref, o_ref):
    dinv = _dinv_col(degp_ref)
    a = jnp.concatenate([acc_ref[0], acc_ref[1]], axis=1)
    y = jnp.maximum(a * dinv + b_ref[...], 0.0)
    xw = jnp.dot(y, w_ref[...], preferred_element_type=jnp.float32)
    z = xw * dinv
    o_ref[0] = z[:, :HALF]
    o_ref[1] = z[:, HALF:]


def _mid(acc1, degp, Wc2, bc1):
    grid = N_PAD // GC_NB
    return pl.pallas_call(
        _mid_body,
        grid=(grid,),
        in_specs=[
            pl.BlockSpec((NSC, GC_NB, HALF), lambda i: (0, i, 0)),
            pl.BlockSpec((NSC, GC_NB, HALF), lambda i: (0, i, 0)),
            pl.BlockSpec((COUT, COUT), lambda i: (0, 0)),
            pl.BlockSpec((1, COUT), lambda i: (0, 0)),
        ],
        out_specs=pl.BlockSpec((NSC, GC_NB, HALF), lambda i: (0, i, 0)),
        out_shape=jax.ShapeDtypeStruct((NSC, N_PAD, HALF), jnp.float32),
    )(acc1, degp, Wc2, bc1[None])


def _fin_body(acc_ref, degp_ref, b_ref, h_ref, o_ref):
    dinv = _dinv_col(degp_ref)
    y = jnp.concatenate([acc_ref[0], acc_ref[1]], axis=1) * dinv + b_ref[...]
    hh = h_ref[...]
    o_ref[...] = jnp.maximum(hh + y, 0.0) + hh


def _fin(acc2, degp, bc2, h_pad):
    grid = N_PAD // GC_NB
    return pl.pallas_call(
        _fin_body,
        grid=(grid,),
        in_specs=[
            pl.BlockSpec((NSC, GC_NB, HALF), lambda i: (0, i, 0)),
            pl.BlockSpec((NSC, GC_NB, HALF), lambda i: (0, i, 0)),
            pl.BlockSpec((1, COUT), lambda i: (0, 0)),
            pl.BlockSpec((GC_NB, COUT), lambda i: (i, 0)),
        ],
        out_specs=pl.BlockSpec((GC_NB, COUT), lambda i: (i, 0)),
        out_shape=jax.ShapeDtypeStruct((N_PAD, COUT), jnp.float32),
    )(acc2, degp, bc2[None], h_pad)


def kernel(x, edge_index, W1, b1, W2, b2, W3, b3, Wc1, bc1, Wc2, bc2):
    src = edge_index[0]
    dst = edge_index[1]
    dst2d = dst.reshape(NCH, C)
    srcoff2d = jnp.concatenate([src, src + N_PAD]).reshape(2 * NCH, C)
    ones2d = jnp.pad(jnp.ones((NSC, N, HALF), jnp.float32),
                     ((0, 0), (0, N_PAD - N), (0, 0))).reshape(NSC * N_PAD, HALF)

    degp = _sc_scatter(ones2d, srcoff2d, dst2d).reshape(NSC, N_PAD, HALF)
    h = _embed(x, W1, b1, W2, b2, W3, b3)
    h_pad = jnp.pad(h, ((0, N_PAD - N), (0, 0)))
    z1 = _pre(h_pad, Wc1, degp)
    acc1 = _sc_scatter(z1.reshape(NSC * N_PAD, HALF), srcoff2d, dst2d)
    z2 = _mid(acc1.reshape(NSC, N_PAD, HALF), degp, Wc2, bc1)
    acc2 = _sc_scatter(z2.reshape(NSC * N_PAD, HALF), srcoff2d, dst2d)
    out = _fin(acc2.reshape(NSC, N_PAD, HALF), degp, bc2, h_pad)
    return out[:N]


# speedup vs baseline: 12.3086x; 1.0744x over previous
import functools

import jax
import jax.numpy as jnp
from jax import lax
from jax.experimental import pallas as pl
from jax.experimental.pallas import tpu as pltpu
from jax.experimental.pallas import tpu_sc as plsc

N = 10000
P = 8
CIN = 512
HID = 512
COUT = 256
HALF = COUT // 2
E = 160000
C = 125
NCH = E // C
NSC = 2
NT = 16
N_PAD = 10240
STRIPE = N_PAD // NT
NBUF = 2
NPHASE = 2


def _vs_mesh():
    return plsc.VectorSubcoreMesh(core_axis_name="c", subcore_axis_name="s")


def _sc_degree(dst2d, ones2d):
    cpt = NCH // (NSC * NT)

    @functools.partial(
        pl.kernel,
        out_type=jax.ShapeDtypeStruct((NSC * N_PAD, HALF), jnp.float32),
        mesh=_vs_mesh(),
        scratch_types=[
            pltpu.VMEM((cpt, C), jnp.int32),
            pltpu.VMEM((128, HALF), jnp.float32),
            pltpu.VMEM_SHARED((N_PAD, HALF), jnp.float32),
            pltpu.SemaphoreType.DMA,
        ],
    )
    def deg_kernel(dst_hbm, ones_hbm, out_hbm, dst_v, ones_v, acc, sem):
        c = lax.axis_index("c")
        s = lax.axis_index("s")
        wid = c * NT + s
        base_n = s * STRIPE
        pltpu.sync_copy(ones_hbm.at[pl.ds(c * N_PAD + base_n, STRIPE)],
                        acc.at[pl.ds(base_n, STRIPE)])
        pltpu.sync_copy(dst_hbm.at[pl.ds(wid * cpt, cpt)], dst_v)
        pltpu.sync_copy(ones_hbm.at[pl.ds(0, 128)], ones_v)
        plsc.subcore_barrier()
        ones_c = ones_v.at[pl.ds(0, C)]

        def fire(j, carry):
            pltpu.async_copy(ones_c, acc.at[dst_v.at[j]], sem, add=True)
            return carry

        lax.fori_loop(0, cpt, fire, 0)

        def drain(j, carry):
            pltpu.make_async_copy(ones_c, acc.at[dst_v.at[0]], sem).wait()
            return carry

        lax.fori_loop(0, cpt, drain, 0)
        plsc.subcore_barrier()
        pltpu.sync_copy(acc.at[pl.ds(base_n, STRIPE)],
                        out_hbm.at[pl.ds(c * N_PAD + base_n, STRIPE)])

    return deg_kernel(dst2d, ones2d)


def _sc_scatter(z2d, srcoff2d, dst2d):
    cpt = NCH // NT
    cpp = cpt // NPHASE

    @functools.partial(
        pl.kernel,
        out_type=jax.ShapeDtypeStruct((NSC * N_PAD, HALF), jnp.float32),
        mesh=_vs_mesh(),
        scratch_types=[
            pltpu.VMEM((cpp, C), jnp.int32),
            pltpu.VMEM((cpp, C), jnp.int32),
            pltpu.VMEM((C, HALF), jnp.float32),
            pltpu.VMEM((C, HALF), jnp.float32),
            pltpu.VMEM_SHARED((N_PAD, HALF), jnp.float32),
            pltpu.SemaphoreType.DMA((NBUF,)),
            pltpu.SemaphoreType.DMA((NBUF,)),
        ],
    )
    def scat_kernel(z_hbm, src_hbm, dst_hbm, out_hbm,
                    src_v, dst_v, buf0, buf1, acc, gsem, ssem):
        c = lax.axis_index("c")
        s = lax.axis_index("s")
        base_n = s * STRIPE
        bufs = [buf0, buf1]
        pltpu.sync_copy(z_hbm.at[pl.ds(c * N_PAD + base_n, STRIPE)],
                        acc.at[pl.ds(base_n, STRIPE)])
        plsc.subcore_barrier()

        for ph in range(NPHASE):
            pltpu.sync_copy(
                src_hbm.at[pl.ds(c * NCH + s * cpt + ph * cpp, cpp)], src_v)
            pltpu.sync_copy(
                dst_hbm.at[pl.ds(s * cpt + ph * cpp, cpp)], dst_v)

            for b in range(NBUF):
                pltpu.async_copy(z_hbm.at[src_v.at[b]], bufs[b], gsem.at[b])

            def round_body(g, carry):
                for b in range(NBUF):
                    j = g * NBUF + b
                    pltpu.make_async_copy(z_hbm.at[src_v.at[j]], bufs[b],
                                          gsem.at[b]).wait()
                    pltpu.async_copy(bufs[b], acc.at[dst_v.at[j]],
                                     ssem.at[b], add=True)
                for b in range(NBUF):
                    j = g * NBUF + b
                    jn = j + NBUF
                    pltpu.make_async_copy(bufs[b], acc.at[dst_v.at[j]],
                                          ssem.at[b]).wait()

                    @pl.when(jn < cpp)
                    def _():
                        pltpu.async_copy(z_hbm.at[src_v.at[jn]], bufs[b],
                                         gsem.at[b])
                return carry

            lax.fori_loop(0, cpp // NBUF, round_body, 0)
        plsc.subcore_barrier()
        pltpu.sync_copy(acc.at[pl.ds(base_n, STRIPE)],
                        out_hbm.at[pl.ds(c * N_PAD + base_n, STRIPE)])

    return scat_kernel(z2d, srcoff2d, dst2d)


EMB_NB = 200
GC_NB = 1024


def _dinv_col(degp_ref):
    deg = degp_ref[0, :, 0:1] + degp_ref[1, :, 0:1] - 1.0
    return lax.rsqrt(jnp.maximum(deg, 1.0))


def _embed_body(x_ref, w1_ref, b1_ref, w2_ref, b2_ref, w3_ref, b3_ref, o_ref):
    xb = x_ref[...].reshape(EMB_NB * P, CIN)
    t = jnp.dot(xb, w1_ref[...], preferred_element_type=jnp.float32)
    t = jnp.maximum(t + b1_ref[...], 0.0)
    sp = jnp.sum(t.reshape(EMB_NB, P, HID), axis=1)
    u = jnp.dot(sp, w2_ref[...], preferred_element_type=jnp.float32)
    u = u + float(P) * b2_ref[...]
    v = jnp.dot(u, w3_ref[...], preferred_element_type=jnp.float32)
    o_ref[...] = jnp.maximum(v + b3_ref[...], 0.0)


def _embed(x, W1, b1, W2, b2, W3, b3):
    grid = N // EMB_NB
    return pl.pallas_call(
        _embed_body,
        grid=(grid,),
        in_specs=[
            pl.BlockSpec((EMB_NB, P, CIN), lambda i: (i, 0, 0)),
            pl.BlockSpec((CIN, HID), lambda i: (0, 0)),
            pl.BlockSpec((1, HID), lambda i: (0, 0)),
            pl.BlockSpec((HID, COUT), lambda i: (0, 0)),
            pl.BlockSpec((1, COUT), lambda i: (0, 0)),
            pl.BlockSpec((COUT, COUT), lambda i: (0, 0)),
            pl.BlockSpec((1, COUT), lambda i: (0, 0)),
        ],
        out_specs=pl.BlockSpec((EMB_NB, COUT), lambda i: (i, 0)),
        out_shape=jax.ShapeDtypeStruct((N, COUT), jnp.float32),
    )(x, W1, b1[None], W2, b2[None], W3, b3[None])


def _pre_body(h_ref, w_ref, degp_ref, o_ref):
    dinv = _dinv_col(degp_ref)
    xw = jnp.dot(h_ref[...], w_ref[...], preferred_element_type=jnp.float32)
    z = xw * dinv
    o_ref[0] = z[:, :HALF]
    o_ref[1] = z[:, HALF:]


def _pre(h_pad, Wc1, degp):
    grid = N_PAD // GC_NB
    return pl.pallas_call(
        _pre_body,
        grid=(grid,),
        in_specs=[
            pl.BlockSpec((GC_NB, COUT), lambda i: (i, 0)),
            pl.BlockSpec((COUT, COUT), lambda i: (0, 0)),
            pl.BlockSpec((NSC, GC_NB, HALF), lambda i: (0, i, 0)),
        ],
        out_specs=pl.BlockSpec((NSC, GC_NB, HALF), lambda i: (0, i, 0)),
        out_shape=jax.ShapeDtypeStruct((NSC, N_PAD, HALF), jnp.float32),
    )(h_pad, Wc1, degp)


def _mid_body(acc_ref, degp_ref, w_ref, b_ref, o_ref):
    dinv = _dinv_col(degp_ref)
    a = jnp.concatenate([acc_ref[0], acc_ref[1]], axis=1)
    y = jnp.maximum(a * dinv + b_ref[...], 0.0)
    xw = jnp.dot(y, w_ref[...], preferred_element_type=jnp.float32)
    z = xw * dinv
    o_ref[0] = z[:, :HALF]
    o_ref[1] = z[:, HALF:]


def _mid(acc1, degp, Wc2, bc1):
    grid = N_PAD // GC_NB
    return pl.pallas_call(
        _mid_body,
        grid=(grid,),
        in_specs=[
            pl.BlockSpec((NSC, GC_NB, HALF), lambda i: (0, i, 0)),
            pl.BlockSpec((NSC, GC_NB, HALF), lambda i: (0, i, 0)),
            pl.BlockSpec((COUT, COUT), lambda i: (0, 0)),
            pl.BlockSpec((1, COUT), lambda i: (0, 0)),
        ],
        out_specs=pl.BlockSpec((NSC, GC_NB, HALF), lambda i: (0, i, 0)),
        out_shape=jax.ShapeDtypeStruct((NSC, N_PAD, HALF), jnp.float32),
    )(acc1, degp, Wc2, bc1[None])


def _fin_body(acc_ref, degp_ref, b_ref, h_ref, o_ref):
    dinv = _dinv_col(degp_ref)
    y = jnp.concatenate([acc_ref[0], acc_ref[1]], axis=1) * dinv + b_ref[...]
    hh = h_ref[...]
    o_ref[...] = jnp.maximum(hh + y, 0.0) + hh


def _fin(acc2, degp, bc2, h_pad):
    grid = N_PAD // GC_NB
    return pl.pallas_call(
        _fin_body,
        grid=(grid,),
        in_specs=[
            pl.BlockSpec((NSC, GC_NB, HALF), lambda i: (0, i, 0)),
            pl.BlockSpec((NSC, GC_NB, HALF), lambda i: (0, i, 0)),
            pl.BlockSpec((1, COUT), lambda i: (0, 0)),
            pl.BlockSpec((GC_NB, COUT), lambda i: (i, 0)),
        ],
        out_specs=pl.BlockSpec((GC_NB, COUT), lambda i: (i, 0)),
        out_shape=jax.ShapeDtypeStruct((N_PAD, COUT), jnp.float32),
    )(acc2, degp, bc2[None], h_pad)


def kernel(x, edge_index, W1, b1, W2, b2, W3, b3, Wc1, bc1, Wc2, bc2):
    src = edge_index[0]
    dst = edge_index[1]
    dst2d = dst.reshape(NCH, C)
    srcoff2d = jnp.concatenate([src, src + N_PAD]).reshape(2 * NCH, C)
    ones2d = jnp.pad(jnp.ones((NSC, N, HALF), jnp.float32),
                     ((0, 0), (0, N_PAD - N), (0, 0))).reshape(NSC * N_PAD, HALF)

    degp = _sc_degree(dst2d, ones2d).reshape(NSC, N_PAD, HALF)
    h = _embed(x, W1, b1, W2, b2, W3, b3)
    h_pad = jnp.pad(h, ((0, N_PAD - N), (0, 0)))
    z1 = _pre(h_pad, Wc1, degp)
    acc1 = _sc_scatter(z1.reshape(NSC * N_PAD, HALF), srcoff2d, dst2d)
    z2 = _mid(acc1.reshape(NSC, N_PAD, HALF), degp, Wc2, bc1)
    acc2 = _sc_scatter(z2.reshape(NSC * N_PAD, HALF), srcoff2d, dst2d)
    out = _fin(acc2.reshape(NSC, N_PAD, HALF), degp, bc2, h_pad)
    return out[:N]


# speedup vs baseline: 12.6176x; 1.0251x over previous
import functools

import jax
import jax.numpy as jnp
from jax import lax
from jax.experimental import pallas as pl
from jax.experimental.pallas import tpu as pltpu
from jax.experimental.pallas import tpu_sc as plsc

N = 10000
P = 8
CIN = 512
HID = 512
COUT = 256
HALF = COUT // 2
E = 160000
C = 125
NCH = E // C
NSC = 2
NT = 16
N_PAD = 10240
STRIPE = N_PAD // NT
NBUF = 2
NPHASE = 2


def _vs_mesh():
    return plsc.VectorSubcoreMesh(core_axis_name="c", subcore_axis_name="s")


def _sc_degree(dst2d, ones2d):
    cpt = NCH // (NSC * NT)

    @functools.partial(
        pl.kernel,
        out_type=jax.ShapeDtypeStruct((NSC * N_PAD, HALF), jnp.float32),
        mesh=_vs_mesh(),
        scratch_types=[
            pltpu.VMEM((cpt, C), jnp.int32),
            pltpu.VMEM((128, HALF), jnp.float32),
            pltpu.VMEM_SHARED((N_PAD, HALF), jnp.float32),
            pltpu.SemaphoreType.DMA,
        ],
    )
    def deg_kernel(dst_hbm, ones_hbm, out_hbm, dst_v, ones_v, acc, sem):
        c = lax.axis_index("c")
        s = lax.axis_index("s")
        wid = c * NT + s
        base_n = s * STRIPE
        pltpu.sync_copy(ones_hbm.at[pl.ds(c * N_PAD + base_n, STRIPE)],
                        acc.at[pl.ds(base_n, STRIPE)])
        pltpu.sync_copy(dst_hbm.at[pl.ds(wid * cpt, cpt)], dst_v)
        pltpu.sync_copy(ones_hbm.at[pl.ds(0, 128)], ones_v)
        plsc.subcore_barrier()
        ones_c = ones_v.at[pl.ds(0, C)]

        def fire(j, carry):
            pltpu.async_copy(ones_c, acc.at[dst_v.at[j]], sem, add=True)
            return carry

        lax.fori_loop(0, cpt, fire, 0)

        def drain(j, carry):
            pltpu.make_async_copy(ones_c, acc.at[dst_v.at[0]], sem).wait()
            return carry

        lax.fori_loop(0, cpt, drain, 0)
        plsc.subcore_barrier()
        pltpu.sync_copy(acc.at[pl.ds(base_n, STRIPE)],
                        out_hbm.at[pl.ds(c * N_PAD + base_n, STRIPE)])

    return deg_kernel(dst2d, ones2d)


def _sc_scatter(z2d, srcoff2d, dst2d):
    cpt = NCH // NT
    cpp = cpt // NPHASE

    @functools.partial(
        pl.kernel,
        out_type=jax.ShapeDtypeStruct((NSC * N_PAD, HALF), jnp.float32),
        mesh=_vs_mesh(),
        scratch_types=[
            pltpu.VMEM((cpp, C), jnp.int32),
            pltpu.VMEM((cpp, C), jnp.int32),
            pltpu.VMEM((C, HALF), jnp.float32),
            pltpu.VMEM((C, HALF), jnp.float32),
            pltpu.VMEM_SHARED((N_PAD, HALF), jnp.float32),
            pltpu.SemaphoreType.DMA((NBUF,)),
            pltpu.SemaphoreType.DMA((NBUF,)),
        ],
    )
    def scat_kernel(z_hbm, src_hbm, dst_hbm, out_hbm,
                    src_v, dst_v, buf0, buf1, acc, gsem, ssem):
        c = lax.axis_index("c")
        s = lax.axis_index("s")
        base_n = s * STRIPE
        bufs = [buf0, buf1]
        pltpu.sync_copy(z_hbm.at[pl.ds(c * N_PAD + base_n, STRIPE)],
                        acc.at[pl.ds(base_n, STRIPE)])
        plsc.subcore_barrier()

        for ph in range(NPHASE):
            pltpu.sync_copy(
                src_hbm.at[pl.ds(c * NCH + s * cpt + ph * cpp, cpp)], src_v)
            pltpu.sync_copy(
                dst_hbm.at[pl.ds(s * cpt + ph * cpp, cpp)], dst_v)

            for b in range(NBUF):
                pltpu.async_copy(z_hbm.at[src_v.at[b]], bufs[b], gsem.at[b])

            def round_body(g, carry):
                for b in range(NBUF):
                    j = g * NBUF + b
                    pltpu.make_async_copy(z_hbm.at[src_v.at[j]], bufs[b],
                                          gsem.at[b]).wait()
                    pltpu.async_copy(bufs[b], acc.at[dst_v.at[j]],
                                     ssem.at[b], add=True)
                for b in range(NBUF):
                    j = g * NBUF + b
                    jn = j + NBUF
                    pltpu.make_async_copy(bufs[b], acc.at[dst_v.at[j]],
                                          ssem.at[b]).wait()

                    @pl.when(jn < cpp)
                    def _():
                        pltpu.async_copy(z_hbm.at[src_v.at[jn]], bufs[b],
                                         gsem.at[b])
                return carry

            lax.fori_loop(0, cpp // NBUF, round_body, 0)
        plsc.subcore_barrier()
        pltpu.sync_copy(acc.at[pl.ds(base_n, STRIPE)],
                        out_hbm.at[pl.ds(c * N_PAD + base_n, STRIPE)])

    return scat_kernel(z2d, srcoff2d, dst2d)


EMB_NB = 400
GC_NB = 1024


def _dinv_col(degp_ref):
    deg = degp_ref[0, :, 0:1] + degp_ref[1, :, 0:1] - 1.0
    return lax.rsqrt(jnp.maximum(deg, 1.0))


def _embed_body(x_ref, w1_ref, b1_ref, w2_ref, b2_ref, w3_ref, b3_ref, o_ref):
    xb = x_ref[...].reshape(EMB_NB * P, CIN)
    t = jnp.dot(xb, w1_ref[...], preferred_element_type=jnp.float32)
    t = jnp.maximum(t + b1_ref[...], 0.0)
    sp = jnp.sum(t.reshape(EMB_NB, P, HID), axis=1)
    u = jnp.dot(sp, w2_ref[...], preferred_element_type=jnp.float32)
    u = u + float(P) * b2_ref[...]
    v = jnp.dot(u, w3_ref[...], preferred_element_type=jnp.float32)
    o_ref[...] = jnp.maximum(v + b3_ref[...], 0.0)


def _embed(x, W1, b1, W2, b2, W3, b3):
    grid = N // EMB_NB
    return pl.pallas_call(
        _embed_body,
        grid=(grid,),
        in_specs=[
            pl.BlockSpec((EMB_NB, P, CIN), lambda i: (i, 0, 0)),
            pl.BlockSpec((CIN, HID), lambda i: (0, 0)),
            pl.BlockSpec((1, HID), lambda i: (0, 0)),
            pl.BlockSpec((HID, COUT), lambda i: (0, 0)),
            pl.BlockSpec((1, COUT), lambda i: (0, 0)),
            pl.BlockSpec((COUT, COUT), lambda i: (0, 0)),
            pl.BlockSpec((1, COUT), lambda i: (0, 0)),
        ],
        out_specs=pl.BlockSpec((EMB_NB, COUT), lambda i: (i, 0)),
        out_shape=jax.ShapeDtypeStruct((N, COUT), jnp.float32),
    )(x, W1, b1[None], W2, b2[None], W3, b3[None])


def _pre_body(h_ref, w_ref, degp_ref, o_ref):
    dinv = _dinv_col(degp_ref)
    xw = jnp.dot(h_ref[...], w_ref[...], preferred_element_type=jnp.float32)
    z = xw * dinv
    o_ref[0] = z[:, :HALF]
    o_ref[1] = z[:, HALF:]


def _pre(h_pad, Wc1, degp):
    grid = N_PAD // GC_NB
    return pl.pallas_call(
        _pre_body,
        grid=(grid,),
        in_specs=[
            pl.BlockSpec((GC_NB, COUT), lambda i: (i, 0)),
            pl.BlockSpec((COUT, COUT), lambda i: (0, 0)),
            pl.BlockSpec((NSC, GC_NB, HALF), lambda i: (0, i, 0)),
        ],
        out_specs=pl.BlockSpec((NSC, GC_NB, HALF), lambda i: (0, i, 0)),
        out_shape=jax.ShapeDtypeStruct((NSC, N_PAD, HALF), jnp.float32),
    )(h_pad, Wc1, degp)


def _mid_body(acc_ref, degp_ref, w_ref, b_ref, o_ref):
    dinv = _dinv_col(degp_ref)
    a = jnp.concatenate([acc_ref[0], acc_ref[1]], axis=1)
    y = jnp.maximum(a * dinv + b_ref[...], 0.0)
    xw = jnp.dot(y, w_ref[...], preferred_element_type=jnp.float32)
    z = xw * dinv
    o_ref[0] = z[:, :HALF]
    o_ref[1] = z[:, HALF:]


def _mid(acc1, degp, Wc2, bc1):
    grid = N_PAD // GC_NB
    return pl.pallas_call(
        _mid_body,
        grid=(grid,),
        in_specs=[
            pl.BlockSpec((NSC, GC_NB, HALF), lambda i: (0, i, 0)),
            pl.BlockSpec((NSC, GC_NB, HALF), lambda i: (0, i, 0)),
            pl.BlockSpec((COUT, COUT), lambda i: (0, 0)),
            pl.BlockSpec((1, COUT), lambda i: (0, 0)),
        ],
        out_specs=pl.BlockSpec((NSC, GC_NB, HALF), lambda i: (0, i, 0)),
        out_shape=jax.ShapeDtypeStruct((NSC, N_PAD, HALF), jnp.float32),
    )(acc1, degp, Wc2, bc1[None])


def _fin_body(acc_ref, degp_ref, b_ref, h_ref, o_ref):
    dinv = _dinv_col(degp_ref)
    y = jnp.concatenate([acc_ref[0], acc_ref[1]], axis=1) * dinv + b_ref[...]
    hh = h_ref[...]
    o_ref[...] = jnp.maximum(hh + y, 0.0) + hh


def _fin(acc2, degp, bc2, h_pad):
    grid = N_PAD // GC_NB
    return pl.pallas_call(
        _fin_body,
        grid=(grid,),
        in_specs=[
            pl.BlockSpec((NSC, GC_NB, HALF), lambda i: (0, i, 0)),
            pl.BlockSpec((NSC, GC_NB, HALF), lambda i: (0, i, 0)),
            pl.BlockSpec((1, COUT), lambda i: (0, 0)),
            pl.BlockSpec((GC_NB, COUT), lambda i: (i, 0)),
        ],
        out_specs=pl.BlockSpec((GC_NB, COUT), lambda i: (i, 0)),
        out_shape=jax.ShapeDtypeStruct((N_PAD, COUT), jnp.float32),
    )(acc2, degp, bc2[None], h_pad)


def kernel(x, edge_index, W1, b1, W2, b2, W3, b3, Wc1, bc1, Wc2, bc2):
    src = edge_index[0]
    dst = edge_index[1]
    dst2d = dst.reshape(NCH, C)
    srcoff2d = jnp.concatenate([src, src + N_PAD]).reshape(2 * NCH, C)
    ones2d = jnp.pad(jnp.ones((NSC, N, HALF), jnp.float32),
                     ((0, 0), (0, N_PAD - N), (0, 0))).reshape(NSC * N_PAD, HALF)

    degp = _sc_degree(dst2d, ones2d).reshape(NSC, N_PAD, HALF)
    h = _embed(x, W1, b1, W2, b2, W3, b3)
    h_pad = jnp.pad(h, ((0, N_PAD - N), (0, 0)))
    z1 = _pre(h_pad, Wc1, degp)
    acc1 = _sc_scatter(z1.reshape(NSC * N_PAD, HALF), srcoff2d, dst2d)
    z2 = _mid(acc1.reshape(NSC, N_PAD, HALF), degp, Wc2, bc1)
    acc2 = _sc_scatter(z2.reshape(NSC * N_PAD, HALF), srcoff2d, dst2d)
    out = _fin(acc2.reshape(NSC, N_PAD, HALF), degp, bc2, h_pad)
    return out[:N]


# speedup vs baseline: 12.6408x; 1.0018x over previous
import functools

import jax
import jax.numpy as jnp
from jax import lax
from jax.experimental import pallas as pl
from jax.experimental.pallas import tpu as pltpu
from jax.experimental.pallas import tpu_sc as plsc

N = 10000
P = 8
CIN = 512
HID = 512
COUT = 256
HALF = COUT // 2
E = 160000
C = 125
NCH = E // C
NSC = 2
NT = 16
N_PAD = 10240
STRIPE = N_PAD // NT
NBUF = 2
NPHASE = 2


def _vs_mesh():
    return plsc.VectorSubcoreMesh(core_axis_name="c", subcore_axis_name="s")


def _sc_degree(dst2d, ones2d):
    cpt = NCH // (NSC * NT)

    @functools.partial(
        pl.kernel,
        out_type=jax.ShapeDtypeStruct((NSC * N_PAD, HALF), jnp.float32),
        mesh=_vs_mesh(),
        scratch_types=[
            pltpu.VMEM((cpt, C), jnp.int32),
            pltpu.VMEM((128, HALF), jnp.float32),
            pltpu.VMEM_SHARED((N_PAD, HALF), jnp.float32),
            pltpu.SemaphoreType.DMA,
        ],
    )
    def deg_kernel(dst_hbm, ones_hbm, out_hbm, dst_v, ones_v, acc, sem):
        c = lax.axis_index("c")
        s = lax.axis_index("s")
        wid = c * NT + s
        base_n = s * STRIPE
        pltpu.sync_copy(ones_hbm.at[pl.ds(c * N_PAD + base_n, STRIPE)],
                        acc.at[pl.ds(base_n, STRIPE)])
        pltpu.sync_copy(dst_hbm.at[pl.ds(wid * cpt, cpt)], dst_v)
        pltpu.sync_copy(ones_hbm.at[pl.ds(0, 128)], ones_v)
        plsc.subcore_barrier()
        ones_c = ones_v.at[pl.ds(0, C)]

        def fire(j, carry):
            pltpu.async_copy(ones_c, acc.at[dst_v.at[j]], sem, add=True)
            return carry

        lax.fori_loop(0, cpt, fire, 0)

        def drain(j, carry):
            pltpu.make_async_copy(ones_c, acc.at[dst_v.at[0]], sem).wait()
            return carry

        lax.fori_loop(0, cpt, drain, 0)
        plsc.subcore_barrier()
        pltpu.sync_copy(acc.at[pl.ds(base_n, STRIPE)],
                        out_hbm.at[pl.ds(c * N_PAD + base_n, STRIPE)])

    return deg_kernel(dst2d, ones2d)


def _sc_scatter(z2d, srcoff2d, dst2d):
    cpt = NCH // NT
    cpp = cpt // NPHASE

    @functools.partial(
        pl.kernel,
        out_type=jax.ShapeDtypeStruct((NSC * N_PAD, HALF), jnp.float32),
        mesh=_vs_mesh(),
        scratch_types=[
            pltpu.VMEM((cpp, C), jnp.int32),
            pltpu.VMEM((cpp, C), jnp.int32),
            pltpu.VMEM((C, HALF), jnp.float32),
            pltpu.VMEM((C, HALF), jnp.float32),
            pltpu.VMEM_SHARED((N_PAD, HALF), jnp.float32),
            pltpu.SemaphoreType.DMA((NBUF,)),
            pltpu.SemaphoreType.DMA((NBUF,)),
        ],
    )
    def scat_kernel(z_hbm, src_hbm, dst_hbm, out_hbm,
                    src_v, dst_v, buf0, buf1, acc, gsem, ssem):
        c = lax.axis_index("c")
        s = lax.axis_index("s")
        base_n = s * STRIPE
        bufs = [buf0, buf1]
        pltpu.sync_copy(z_hbm.at[pl.ds(c * N_PAD + base_n, STRIPE)],
                        acc.at[pl.ds(base_n, STRIPE)])
        plsc.subcore_barrier()

        for ph in range(NPHASE):
            pltpu.sync_copy(
                src_hbm.at[pl.ds(c * NCH + s * cpt + ph * cpp, cpp)], src_v)
            pltpu.sync_copy(
                dst_hbm.at[pl.ds(s * cpt + ph * cpp, cpp)], dst_v)

            for b in range(NBUF):
                pltpu.async_copy(z_hbm.at[src_v.at[b]], bufs[b], gsem.at[b])

            def round_body(g, carry):
                for b in range(NBUF):
                    j = g * NBUF + b
                    pltpu.make_async_copy(z_hbm.at[src_v.at[j]], bufs[b],
                                          gsem.at[b]).wait()
                    pltpu.async_copy(bufs[b], acc.at[dst_v.at[j]],
                                     ssem.at[b], add=True)
                for b in range(NBUF):
                    j = g * NBUF + b
                    jn = j + NBUF
                    pltpu.make_async_copy(bufs[b], acc.at[dst_v.at[j]],
                                          ssem.at[b]).wait()

                    @pl.when(jn < cpp)
                    def _():
                        pltpu.async_copy(z_hbm.at[src_v.at[jn]], bufs[b],
                                         gsem.at[b])
                return carry

            lax.fori_loop(0, cpp // NBUF, round_body, 0)
        plsc.subcore_barrier()
        pltpu.sync_copy(acc.at[pl.ds(base_n, STRIPE)],
                        out_hbm.at[pl.ds(c * N_PAD + base_n, STRIPE)])

    return scat_kernel(z2d, srcoff2d, dst2d)


EMB_NB = 400
GC_NB = 1024


DEGW = 8


def _dinv_col(degp_ref):
    deg = degp_ref[0, :, 0:1] + degp_ref[1, :, 0:1] - 1.0
    return lax.rsqrt(jnp.maximum(deg, 1.0))


def _embed_body(x_ref, w1_ref, b1_ref, w2_ref, b2_ref, w3_ref, b3_ref, o_ref):
    xb = x_ref[...].reshape(EMB_NB * P, CIN)
    t = jnp.dot(xb, w1_ref[...], preferred_element_type=jnp.float32)
    t = jnp.maximum(t + b1_ref[...], 0.0)
    sp = jnp.sum(t.reshape(EMB_NB, P, HID), axis=1)
    u = jnp.dot(sp, w2_ref[...], preferred_element_type=jnp.float32)
    u = u + float(P) * b2_ref[...]
    v = jnp.dot(u, w3_ref[...], preferred_element_type=jnp.float32)
    o_ref[...] = jnp.maximum(v + b3_ref[...], 0.0)


def _embed(x, W1, b1, W2, b2, W3, b3):
    grid = N // EMB_NB
    return pl.pallas_call(
        _embed_body,
        grid=(grid,),
        in_specs=[
            pl.BlockSpec((EMB_NB, P, CIN), lambda i: (i, 0, 0)),
            pl.BlockSpec((CIN, HID), lambda i: (0, 0)),
            pl.BlockSpec((1, HID), lambda i: (0, 0)),
            pl.BlockSpec((HID, COUT), lambda i: (0, 0)),
            pl.BlockSpec((1, COUT), lambda i: (0, 0)),
            pl.BlockSpec((COUT, COUT), lambda i: (0, 0)),
            pl.BlockSpec((1, COUT), lambda i: (0, 0)),
        ],
        out_specs=pl.BlockSpec((EMB_NB, COUT), lambda i: (i, 0)),
        out_shape=jax.ShapeDtypeStruct((N, COUT), jnp.float32),
    )(x, W1, b1[None], W2, b2[None], W3, b3[None])


def _pre_body(h_ref, w_ref, degp_ref, o_ref):
    dinv = _dinv_col(degp_ref)
    xw = jnp.dot(h_ref[...], w_ref[...], preferred_element_type=jnp.float32)
    z = xw * dinv
    o_ref[0] = z[:, :HALF]
    o_ref[1] = z[:, HALF:]


def _pre(h_pad, Wc1, degp):
    grid = N_PAD // GC_NB
    return pl.pallas_call(
        _pre_body,
        grid=(grid,),
        in_specs=[
            pl.BlockSpec((GC_NB, COUT), lambda i: (i, 0)),
            pl.BlockSpec((COUT, COUT), lambda i: (0, 0)),
            pl.BlockSpec((NSC, GC_NB, DEGW), lambda i: (0, i, 0)),
        ],
        out_specs=pl.BlockSpec((NSC, GC_NB, HALF), lambda i: (0, i, 0)),
        out_shape=jax.ShapeDtypeStruct((NSC, N_PAD, HALF), jnp.float32),
    )(h_pad, Wc1, degp)


def _mid_body(acc_ref, degp_ref, w_ref, b_ref, o_ref):
    dinv = _dinv_col(degp_ref)
    a = jnp.concatenate([acc_ref[0], acc_ref[1]], axis=1)
    y = jnp.maximum(a * dinv + b_ref[...], 0.0)
    xw = jnp.dot(y, w_ref[...], preferred_element_type=jnp.float32)
    z = xw * dinv
    o_ref[0] = z[:, :HALF]
    o_ref[1] = z[:, HALF:]


def _mid(acc1, degp, Wc2, bc1):
    grid = N_PAD // GC_NB
    return pl.pallas_call(
        _mid_body,
        grid=(grid,),
        in_specs=[
            pl.BlockSpec((NSC, GC_NB, HALF), lambda i: (0, i, 0)),
            pl.BlockSpec((NSC, GC_NB, DEGW), lambda i: (0, i, 0)),
            pl.BlockSpec((COUT, COUT), lambda i: (0, 0)),
            pl.BlockSpec((1, COUT), lambda i: (0, 0)),
        ],
        out_specs=pl.BlockSpec((NSC, GC_NB, HALF), lambda i: (0, i, 0)),
        out_shape=jax.ShapeDtypeStruct((NSC, N_PAD, HALF), jnp.float32),
    )(acc1, degp, Wc2, bc1[None])


def _fin_body(acc_ref, degp_ref, b_ref, h_ref, o_ref):
    dinv = _dinv_col(degp_ref)
    y = jnp.concatenate([acc_ref[0], acc_ref[1]], axis=1) * dinv + b_ref[...]
    hh = h_ref[...]
    o_ref[...] = jnp.maximum(hh + y, 0.0) + hh


def _fin(acc2, degp, bc2, h_pad):
    grid = N_PAD // GC_NB
    return pl.pallas_call(
        _fin_body,
        grid=(grid,),
        in_specs=[
            pl.BlockSpec((NSC, GC_NB, HALF), lambda i: (0, i, 0)),
            pl.BlockSpec((NSC, GC_NB, DEGW), lambda i: (0, i, 0)),
            pl.BlockSpec((1, COUT), lambda i: (0, 0)),
            pl.BlockSpec((GC_NB, COUT), lambda i: (i, 0)),
        ],
        out_specs=pl.BlockSpec((GC_NB, COUT), lambda i: (i, 0)),
        out_shape=jax.ShapeDtypeStruct((N_PAD, COUT), jnp.float32),
    )(acc2, degp, bc2[None], h_pad)


def kernel(x, edge_index, W1, b1, W2, b2, W3, b3, Wc1, bc1, Wc2, bc2):
    src = edge_index[0]
    dst = edge_index[1]
    dst2d = dst.reshape(NCH, C)
    srcoff2d = jnp.concatenate([src, src + N_PAD]).reshape(2 * NCH, C)
    ones2d = jnp.pad(jnp.ones((NSC, N, HALF), jnp.float32),
                     ((0, 0), (0, N_PAD - N), (0, 0))).reshape(NSC * N_PAD, HALF)

    degp = _sc_degree(dst2d, ones2d).reshape(NSC, N_PAD, HALF)[:, :, :DEGW]
    h = _embed(x, W1, b1, W2, b2, W3, b3)
    h_pad = jnp.pad(h, ((0, N_PAD - N), (0, 0)))
    z1 = _pre(h_pad, Wc1, degp)
    acc1 = _sc_scatter(z1.reshape(NSC * N_PAD, HALF), srcoff2d, dst2d)
    z2 = _mid(acc1.reshape(NSC, N_PAD, HALF), degp, Wc2, bc1)
    acc2 = _sc_scatter(z2.reshape(NSC * N_PAD, HALF), srcoff2d, dst2d)
    out = _fin(acc2.reshape(NSC, N_PAD, HALF), degp, bc2, h_pad)
    return out[:N]
